# Initial kernel scaffold; baseline (speedup 1.0000x reference)
import jax
import jax.numpy as jnp
from jax.experimental import pallas as pl


def kernel(x, W, H):
    raise NotImplementedError("write your pallas kernel here")


# speedup vs baseline: 1.5265x; 1.5265x over previous
import functools

import jax
import jax.numpy as jnp
from jax import lax
from jax.experimental import pallas as pl
from jax.experimental.pallas import tpu as pltpu
from jax.experimental.pallas import tpu_sc as plsc

B = 16384
EMB = 128
NC = 2
NS = 16
NW = NC * NS
BPW = B // NW
CH = 128
NCH = BPW // CH
GRP = CH // 16


def _mf_body(uid_hbm, iid_hbm, w_hbm, h_hbm, out_hbm,
             uid_v, iid_v, wb0, wb1, hb0, hb1, out_v,
             sw0, sw1, sh0, sh1):
    wid = lax.axis_index("s") * NC + lax.axis_index("c")
    base = wid * BPW

    pltpu.sync_copy(uid_hbm.at[pl.ds(wid * NCH, NCH)], uid_v)
    pltpu.sync_copy(iid_hbm.at[pl.ds(wid * NCH, NCH)], iid_v)

    wbufs = (wb0, wb1)
    hbufs = (hb0, hb1)
    wsems = (sw0, sw1)
    hsems = (sh0, sh1)

    def start(c):
        slot = c % 2
        cw = pltpu.async_copy(w_hbm.at[uid_v.at[c]], wbufs[slot], wsems[slot])
        chh = pltpu.async_copy(h_hbm.at[iid_v.at[c]], hbufs[slot], hsems[slot])
        return cw, chh

    lane = lax.iota(jnp.int32, 16)
    perms = [lane ^ st for st in (8, 4, 2, 1)]
    inflight = {0: start(0)}

    for c in range(NCH):
        if c + 1 < NCH:
            inflight[c + 1] = start(c + 1)
        for h in inflight.pop(c):
            h.wait()
        slot = c % 2
        wref = wbufs[slot]
        href = hbufs[slot]

        def group_body(g, _, wref=wref, href=href, c=c):
            row0 = g * 16

            def row_body(r, res):
                row = row0 + r
                acc = None
                for j in range(EMB // 16):
                    w = wref[row, pl.ds(j * 16, 16)]
                    h = href[row, pl.ds(j * 16, 16)]
                    p = w * h
                    acc = p if acc is None else acc + p
                for perm in perms:
                    acc = acc + acc.at[perm].get(mode="promise_in_bounds")
                return jnp.where(lane == r, acc, res)

            res = lax.fori_loop(0, 16, row_body, jnp.zeros((16,), jnp.float32))
            pred = 1.0 / (1.0 + jnp.exp(-res))
            out_v[pl.ds(c * CH + row0, 16)] = pred
            return 0

        lax.fori_loop(0, GRP, group_body, 0)

    pltpu.sync_copy(out_v, out_hbm.at[pl.ds(base, BPW)])


@jax.jit
def kernel(x, W, H):
    uidx = x[:, 0].reshape(NW * NCH, CH)
    iidx = x[:, 1].reshape(NW * NCH, CH)
    mesh = plsc.VectorSubcoreMesh(core_axis_name="c", subcore_axis_name="s")
    f = pl.kernel(
        _mf_body,
        out_type=jax.ShapeDtypeStruct((B,), jnp.float32),
        mesh=mesh,
        scratch_types=[
            pltpu.VMEM((NCH, CH), jnp.int32),
            pltpu.VMEM((NCH, CH), jnp.int32),
            pltpu.VMEM((CH, EMB), jnp.float32),
            pltpu.VMEM((CH, EMB), jnp.float32),
            pltpu.VMEM((CH, EMB), jnp.float32),
            pltpu.VMEM((CH, EMB), jnp.float32),
            pltpu.VMEM((BPW,), jnp.float32),
            pltpu.SemaphoreType.DMA,
            pltpu.SemaphoreType.DMA,
            pltpu.SemaphoreType.DMA,
            pltpu.SemaphoreType.DMA,
        ],
    )
    return f(uidx, iidx, W, H)
